# Initial kernel scaffold; baseline (speedup 1.0000x reference)
import jax
import jax.numpy as jnp
from jax.experimental import pallas as pl


def kernel(x, adj, W0, b0, gamma0, beta0):
    raise NotImplementedError("write your pallas kernel here")


# speedup vs baseline: 4.3178x; 4.3178x over previous
import functools

import jax
import jax.numpy as jnp
from jax.experimental import pallas as pl
from jax.experimental.pallas import tpu as pltpu


def _matmul_kernel(adj_ref, x_ref, w_ref, b_ref, h_ref, s_ref, ss_ref, *, tm):
    i = pl.program_id(0)
    g = jnp.dot(adj_ref[...], x_ref[...], preferred_element_type=jnp.float32)
    g = g + x_ref[pl.ds(i * tm, tm), :]
    h = jnp.dot(g, w_ref[...],
                preferred_element_type=jnp.float32) + b_ref[0:1, :]
    h_ref[...] = h

    @pl.when(i == 0)
    def _():
        s_ref[...] = jnp.zeros_like(s_ref)
        ss_ref[...] = jnp.zeros_like(ss_ref)

    s_ref[0:1, :] += jnp.sum(h, axis=0, keepdims=True)
    ss_ref[0:1, :] += jnp.sum(h * h, axis=0, keepdims=True)


def _norm_kernel(h_ref, x_ref, s_ref, ss_ref, gam_ref, bet_ref, o_ref, *, n):
    mean = s_ref[0:1, :] / n
    var = ss_ref[0:1, :] / n - mean * mean
    inv = jax.lax.rsqrt(var + 1e-5)
    t = (h_ref[...] - mean) * (inv * gam_ref[0:1, :]) + bet_ref[0:1, :]
    o_ref[...] = jnp.where(t >= 0, t, 0.01 * t) + x_ref[...]


def kernel(x, adj, W0, b0, gamma0, beta0):
    n, d = x.shape
    tm = 80
    ni = n // tm

    b = b0.reshape(1, d)
    gam = gamma0.reshape(1, d)
    bet = beta0.reshape(1, d)

    h, s, ss = pl.pallas_call(
        functools.partial(_matmul_kernel, tm=tm),
        grid=(ni,),
        in_specs=[
            pl.BlockSpec((tm, n), lambda i: (i, 0)),
            pl.BlockSpec((n, d), lambda i: (0, 0)),
            pl.BlockSpec((d, d), lambda i: (0, 0)),
            pl.BlockSpec((1, d), lambda i: (0, 0)),
        ],
        out_specs=[
            pl.BlockSpec((tm, d), lambda i: (i, 0)),
            pl.BlockSpec((8, d), lambda i: (0, 0)),
            pl.BlockSpec((8, d), lambda i: (0, 0)),
        ],
        out_shape=[
            jax.ShapeDtypeStruct((n, d), jnp.float32),
            jax.ShapeDtypeStruct((8, d), jnp.float32),
            jax.ShapeDtypeStruct((8, d), jnp.float32),
        ],
        compiler_params=pltpu.CompilerParams(
            dimension_semantics=("arbitrary",)),
    )(adj, x, W0, b)

    tn = 1000
    out = pl.pallas_call(
        functools.partial(_norm_kernel, n=float(n)),
        grid=(n // tn,),
        in_specs=[
            pl.BlockSpec((tn, d), lambda i: (i, 0)),
            pl.BlockSpec((tn, d), lambda i: (i, 0)),
            pl.BlockSpec((8, d), lambda i: (0, 0)),
            pl.BlockSpec((8, d), lambda i: (0, 0)),
            pl.BlockSpec((1, d), lambda i: (0, 0)),
            pl.BlockSpec((1, d), lambda i: (0, 0)),
        ],
        out_specs=pl.BlockSpec((tn, d), lambda i: (i, 0)),
        out_shape=jax.ShapeDtypeStruct((n, d), jnp.float32),
    )(h, x, s, ss, gam, bet)
    return out


# speedup vs baseline: 5.5525x; 1.2859x over previous
import functools

import jax
import jax.numpy as jnp
from jax.experimental import pallas as pl
from jax.experimental.pallas import tpu as pltpu


def _matmul_kernel(adja_ref, adjb_ref, x_ref, w_ref, b_ref,
                   h_ref, s_ref, ss_ref, *, tm):
    i = pl.program_id(0)

    @pl.when(i == 0)
    def _():
        s_ref[...] = jnp.zeros_like(s_ref)
        ss_ref[...] = jnp.zeros_like(ss_ref)

    ra = (2 * i) * tm
    ga = jnp.dot(adja_ref[...], x_ref[...],
                 preferred_element_type=jnp.float32)
    ga = ga + x_ref[pl.ds(ra, tm), :]
    ha = jnp.dot(ga, w_ref[...],
                 preferred_element_type=jnp.float32) + b_ref[0:1, :]
    h_ref[0:tm, :] = ha

    gb = jnp.dot(adjb_ref[...], x_ref[...],
                 preferred_element_type=jnp.float32)
    gb = gb + x_ref[pl.ds(ra + tm, tm), :]
    hb = jnp.dot(gb, w_ref[...],
                 preferred_element_type=jnp.float32) + b_ref[0:1, :]
    h_ref[tm:2 * tm, :] = hb

    s_ref[0:1, :] += (jnp.sum(ha, axis=0, keepdims=True) +
                      jnp.sum(hb, axis=0, keepdims=True))
    ss_ref[0:1, :] += (jnp.sum(ha * ha, axis=0, keepdims=True) +
                       jnp.sum(hb * hb, axis=0, keepdims=True))


def _norm_kernel(h_ref, x_ref, s_ref, ss_ref, gam_ref, bet_ref, o_ref, *, n):
    mean = s_ref[0:1, :] / n
    var = ss_ref[0:1, :] / n - mean * mean
    inv = jax.lax.rsqrt(var + 1e-5)
    t = (h_ref[...] - mean) * (inv * gam_ref[0:1, :]) + bet_ref[0:1, :]
    o_ref[...] = jnp.where(t >= 0, t, 0.01 * t) + x_ref[...]


def kernel(x, adj, W0, b0, gamma0, beta0):
    n, d = x.shape
    tm = 200
    ni = n // (2 * tm)

    b = b0.reshape(1, d)
    gam = gamma0.reshape(1, d)
    bet = beta0.reshape(1, d)

    h, s, ss = pl.pallas_call(
        functools.partial(_matmul_kernel, tm=tm),
        grid=(ni,),
        in_specs=[
            pl.BlockSpec((tm, n), lambda i: (2 * i, 0)),
            pl.BlockSpec((tm, n), lambda i: (2 * i + 1, 0)),
            pl.BlockSpec((n, d), lambda i: (0, 0)),
            pl.BlockSpec((d, d), lambda i: (0, 0)),
            pl.BlockSpec((1, d), lambda i: (0, 0)),
        ],
        out_specs=[
            pl.BlockSpec((2 * tm, d), lambda i: (i, 0)),
            pl.BlockSpec((8, d), lambda i: (0, 0)),
            pl.BlockSpec((8, d), lambda i: (0, 0)),
        ],
        out_shape=[
            jax.ShapeDtypeStruct((n, d), jnp.float32),
            jax.ShapeDtypeStruct((8, d), jnp.float32),
            jax.ShapeDtypeStruct((8, d), jnp.float32),
        ],
        compiler_params=pltpu.CompilerParams(
            dimension_semantics=("arbitrary",)),
    )(adj, adj, x, W0, b)

    tn = 1000
    out = pl.pallas_call(
        functools.partial(_norm_kernel, n=float(n)),
        grid=(n // tn,),
        in_specs=[
            pl.BlockSpec((tn, d), lambda i: (i, 0)),
            pl.BlockSpec((tn, d), lambda i: (i, 0)),
            pl.BlockSpec((8, d), lambda i: (0, 0)),
            pl.BlockSpec((8, d), lambda i: (0, 0)),
            pl.BlockSpec((1, d), lambda i: (0, 0)),
            pl.BlockSpec((1, d), lambda i: (0, 0)),
        ],
        out_specs=pl.BlockSpec((tn, d), lambda i: (i, 0)),
        out_shape=jax.ShapeDtypeStruct((n, d), jnp.float32),
    )(h, x, s, ss, gam, bet)
    return out


# speedup vs baseline: 6.0077x; 1.0820x over previous
import functools

import jax
import jax.numpy as jnp
from jax.experimental import pallas as pl
from jax.experimental.pallas import tpu as pltpu


def _matmul_kernel(adj_ref, x_ref, w_ref, b_ref, h_ref, s_ref, *, tm):
    i = pl.program_id(0)
    g = jnp.dot(adj_ref[...], x_ref[...], preferred_element_type=jnp.float32)
    g = g + x_ref[pl.ds(i * tm, tm), :]
    h = jnp.dot(g, w_ref[...],
                preferred_element_type=jnp.float32) + b_ref[0:1, :]
    h_ref[...] = h
    s_ref[0, 0:1, :] = jnp.sum(h, axis=0, keepdims=True)
    s_ref[0, 1:2, :] = jnp.sum(h * h, axis=0, keepdims=True)


def _norm_kernel(h_ref, x_ref, s_ref, gam_ref, bet_ref, o_ref, *, n):
    mean = jnp.sum(s_ref[:, 0, :], axis=0, keepdims=True) / n
    msq = jnp.sum(s_ref[:, 1, :], axis=0, keepdims=True) / n
    var = msq - mean * mean
    inv = jax.lax.rsqrt(var + 1e-5)
    t = (h_ref[...] - mean) * (inv * gam_ref[0:1, :]) + bet_ref[0:1, :]
    o_ref[...] = jnp.where(t >= 0, t, 0.01 * t) + x_ref[...]


def kernel(x, adj, W0, b0, gamma0, beta0):
    n, d = x.shape
    tm = 400
    ni = n // tm

    b = b0.reshape(1, d)
    gam = gamma0.reshape(1, d)
    bet = beta0.reshape(1, d)

    h, s = pl.pallas_call(
        functools.partial(_matmul_kernel, tm=tm),
        grid=(ni,),
        in_specs=[
            pl.BlockSpec((tm, n), lambda i: (i, 0)),
            pl.BlockSpec((n, d), lambda i: (0, 0)),
            pl.BlockSpec((d, d), lambda i: (0, 0)),
            pl.BlockSpec((1, d), lambda i: (0, 0)),
        ],
        out_specs=[
            pl.BlockSpec((tm, d), lambda i: (i, 0)),
            pl.BlockSpec((1, 8, d), lambda i: (i, 0, 0)),
        ],
        out_shape=[
            jax.ShapeDtypeStruct((n, d), jnp.float32),
            jax.ShapeDtypeStruct((ni, 8, d), jnp.float32),
        ],
        compiler_params=pltpu.CompilerParams(
            dimension_semantics=("parallel",)),
    )(adj, x, W0, b)

    tn = 1000
    out = pl.pallas_call(
        functools.partial(_norm_kernel, n=float(n)),
        grid=(n // tn,),
        in_specs=[
            pl.BlockSpec((tn, d), lambda i: (i, 0)),
            pl.BlockSpec((tn, d), lambda i: (i, 0)),
            pl.BlockSpec((ni, 8, d), lambda i: (0, 0, 0)),
            pl.BlockSpec((1, d), lambda i: (0, 0)),
            pl.BlockSpec((1, d), lambda i: (0, 0)),
        ],
        out_specs=pl.BlockSpec((tn, d), lambda i: (i, 0)),
        out_shape=jax.ShapeDtypeStruct((n, d), jnp.float32),
        compiler_params=pltpu.CompilerParams(
            dimension_semantics=("parallel",)),
    )(h, x, s, gam, bet)
    return out


# speedup vs baseline: 6.1737x; 1.0276x over previous
import functools

import jax
import jax.numpy as jnp
from jax.experimental import pallas as pl
from jax.experimental.pallas import tpu as pltpu


def _matmul_kernel(adj_ref, x_ref, w_ref, b_ref, h_ref, s_ref, *, tm):
    i = pl.program_id(0)
    g = jnp.dot(adj_ref[...], x_ref[...], preferred_element_type=jnp.float32)
    g = g + x_ref[pl.ds(i * tm, tm), :]
    h = jnp.dot(g, w_ref[...],
                preferred_element_type=jnp.float32) + b_ref[0:1, :]
    h_ref[...] = h.astype(jnp.bfloat16)
    s_ref[0, 0:1, :] = jnp.sum(h, axis=0, keepdims=True)
    s_ref[0, 1:2, :] = jnp.sum(h * h, axis=0, keepdims=True)


def _norm_kernel(h_ref, x_ref, s_ref, gam_ref, bet_ref, o_ref, *, n):
    mean = jnp.sum(s_ref[:, 0, :], axis=0, keepdims=True) / n
    msq = jnp.sum(s_ref[:, 1, :], axis=0, keepdims=True) / n
    var = msq - mean * mean
    inv = jax.lax.rsqrt(var + 1e-5)
    t = (h_ref[...].astype(jnp.float32) - mean) * (inv * gam_ref[0:1, :]) \
        + bet_ref[0:1, :]
    o_ref[...] = jnp.where(t >= 0, t, 0.01 * t) + x_ref[...]


def kernel(x, adj, W0, b0, gamma0, beta0):
    n, d = x.shape
    tm = 400
    ni = n // tm

    b = b0.reshape(1, d)
    gam = gamma0.reshape(1, d)
    bet = beta0.reshape(1, d)

    h, s = pl.pallas_call(
        functools.partial(_matmul_kernel, tm=tm),
        grid=(ni,),
        in_specs=[
            pl.BlockSpec((tm, n), lambda i: (i, 0)),
            pl.BlockSpec((n, d), lambda i: (0, 0)),
            pl.BlockSpec((d, d), lambda i: (0, 0)),
            pl.BlockSpec((1, d), lambda i: (0, 0)),
        ],
        out_specs=[
            pl.BlockSpec((tm, d), lambda i: (i, 0)),
            pl.BlockSpec((1, 8, d), lambda i: (i, 0, 0)),
        ],
        out_shape=[
            jax.ShapeDtypeStruct((n, d), jnp.bfloat16),
            jax.ShapeDtypeStruct((ni, 8, d), jnp.float32),
        ],
        compiler_params=pltpu.CompilerParams(
            dimension_semantics=("parallel",)),
    )(adj, x, W0, b)

    tn = 2000
    out = pl.pallas_call(
        functools.partial(_norm_kernel, n=float(n)),
        grid=(n // tn,),
        in_specs=[
            pl.BlockSpec((tn, d), lambda i: (i, 0)),
            pl.BlockSpec((tn, d), lambda i: (i, 0)),
            pl.BlockSpec((ni, 8, d), lambda i: (0, 0, 0)),
            pl.BlockSpec((1, d), lambda i: (0, 0)),
            pl.BlockSpec((1, d), lambda i: (0, 0)),
        ],
        out_specs=pl.BlockSpec((tn, d), lambda i: (i, 0)),
        out_shape=jax.ShapeDtypeStruct((n, d), jnp.float32),
        compiler_params=pltpu.CompilerParams(
            dimension_semantics=("parallel",)),
    )(h, x, s, gam, bet)
    return out
